# baseline (speedup 1.0000x reference)
import functools

import jax
import jax.numpy as jnp
from jax import lax
from jax.experimental import pallas as pl
from jax.experimental.pallas import tpu as pltpu
from jax.experimental.pallas import tpu_sc as plsc

_H_RATIO = 0.33
_L = 16
_NC = 2
_NS = 16
_CC = 128


def _sc_body(x_hbm, o_hbm, xbuf, ebuf, msmem, *, b, c, h, w, rh):
    hw = h * w
    nv = hw // _L
    nch = c // _CC
    nw = _NC * _NS
    spt = b // nw

    wid = lax.axis_index("s") * _NC + lax.axis_index("c")

    def per_sample(si, _carry):
        smp = wid * spt + si

        def chunk_a(ci, accs):
            pltpu.sync_copy(x_hbm.at[smp, pl.ds(ci * _CC, _CC)], xbuf)

            def row_a(r, a2):
                out = []
                for k in range(nv):
                    x16 = xbuf[r, pl.ds(k * _L, _L)]
                    out.append(a2[k] + x16 * x16)
                return tuple(out)

            return lax.fori_loop(0, _CC, row_a, accs)

        accs0 = tuple(jnp.zeros((_L,), jnp.float32) for _ in range(nv))
        accs = lax.fori_loop(0, nch, chunk_a, accs0)

        for k in range(nv):
            ebuf[k] = accs[k]

        for g in range(h):
            row = ebuf[(g * w) // _L]
            base = (g * w) % _L
            mg = row[base]
            for j in range(1, w):
                mg = jnp.maximum(mg, row[base + j])
            msmem[g] = mg

        keepf = []
        for g in range(h):
            mg = msmem[g]

            def rank_body(j, acc):
                mj = msmem[j]
                beat = (mj > mg) | ((mj == mg) & (j > g))
                return acc + beat.astype(jnp.int32)

            rank = lax.fori_loop(0, h, rank_body, jnp.int32(0))
            keepf.append((rank >= rh).astype(jnp.float32))

        li = lax.iota(jnp.int32, _L)
        maskv = []
        for k in range(nv):
            ka = jnp.full((_L,), keepf[(k * _L) // w], jnp.float32)
            kb = jnp.full((_L,), keepf[(k * _L) // w + 1], jnp.float32)
            maskv.append(jnp.where(li < w, ka, kb))

        def chunk_b(ci, _c2):
            pltpu.sync_copy(x_hbm.at[smp, pl.ds(ci * _CC, _CC)], xbuf)

            def row_b(r, _c3):
                for k in range(nv):
                    xbuf[r, pl.ds(k * _L, _L)] = (
                        xbuf[r, pl.ds(k * _L, _L)] * maskv[k])
                return _c3

            lax.fori_loop(0, _CC, row_b, 0)
            pltpu.sync_copy(xbuf, o_hbm.at[smp, pl.ds(ci * _CC, _CC)])
            return _c2

        lax.fori_loop(0, nch, chunk_b, 0)
        return _carry

    lax.fori_loop(0, spt, per_sample, 0)


def kernel(x):
    b, c, h, w = x.shape
    rh = int(round(_H_RATIO * h))
    hw = h * w
    x3 = x.reshape(b, c, hw)

    mesh = plsc.VectorSubcoreMesh(
        core_axis_name="c", subcore_axis_name="s",
        num_cores=_NC, num_subcores=_NS)
    body = functools.partial(_sc_body, b=b, c=c, h=h, w=w, rh=rh)
    run = pl.kernel(
        body,
        out_type=jax.ShapeDtypeStruct((b, c, hw), x.dtype),
        mesh=mesh,
        scratch_types=[
            pltpu.VMEM((_CC, hw), jnp.float32),
            pltpu.VMEM((hw // _L, _L), jnp.float32),
            pltpu.SMEM((h,), jnp.float32),
        ],
    )
    out = run(x3)
    return out.reshape(b, c, h, w)


# speedup vs baseline: 1.0442x; 1.0442x over previous
import functools

import jax
import jax.numpy as jnp
from jax import lax
from jax.experimental import pallas as pl
from jax.experimental.pallas import tpu as pltpu
from jax.experimental.pallas import tpu_sc as plsc

_H_RATIO = 0.33
_L = 16
_NC = 2
_NS = 16
_CC = 256


def _sc_body(x_hbm, o_hbm, xbuf, ebuf, msmem, *, b, c, h, w, rh):
    hw = h * w
    nv = hw // _L
    nch = c // _CC
    nw = _NC * _NS
    spt = b // nw

    wid = lax.axis_index("s") * _NC + lax.axis_index("c")

    def per_sample(si, _carry):
        smp = wid * spt + si

        def chunk_a(ci, accs):
            pltpu.sync_copy(x_hbm.at[smp, pl.ds(ci * _CC, _CC)], xbuf)

            def row_a(r, a2):
                out = []
                for k in range(nv):
                    x16 = xbuf[r, pl.ds(k * _L, _L)]
                    out.append(a2[k] + x16 * x16)
                return tuple(out)

            return lax.fori_loop(0, _CC, row_a, accs, unroll=8)

        accs0 = tuple(jnp.zeros((_L,), jnp.float32) for _ in range(nv))
        accs = lax.fori_loop(0, nch, chunk_a, accs0)

        for k in range(nv):
            ebuf[k] = accs[k]

        for g in range(h):
            row = ebuf[(g * w) // _L]
            base = (g * w) % _L
            mg = row[base]
            for j in range(1, w):
                mg = jnp.maximum(mg, row[base + j])
            msmem[g] = mg

        keepf = []
        for g in range(h):
            mg = msmem[g]

            def rank_body(j, acc):
                mj = msmem[j]
                beat = (mj > mg) | ((mj == mg) & (j > g))
                return acc + beat.astype(jnp.int32)

            rank = lax.fori_loop(0, h, rank_body, jnp.int32(0))
            keepf.append((rank >= rh).astype(jnp.float32))

        li = lax.iota(jnp.int32, _L)
        maskv = []
        for k in range(nv):
            ka = jnp.full((_L,), keepf[(k * _L) // w], jnp.float32)
            kb = jnp.full((_L,), keepf[(k * _L) // w + 1], jnp.float32)
            maskv.append(jnp.where(li < w, ka, kb))

        def chunk_b(ci, _c2):
            pltpu.sync_copy(x_hbm.at[smp, pl.ds(ci * _CC, _CC)], xbuf)

            def row_b(r, _c3):
                for k in range(nv):
                    xbuf[r, pl.ds(k * _L, _L)] = (
                        xbuf[r, pl.ds(k * _L, _L)] * maskv[k])
                return _c3

            lax.fori_loop(0, _CC, row_b, 0, unroll=8)
            pltpu.sync_copy(xbuf, o_hbm.at[smp, pl.ds(ci * _CC, _CC)])
            return _c2

        lax.fori_loop(0, nch, chunk_b, 0)
        return _carry

    lax.fori_loop(0, spt, per_sample, 0)


def kernel(x):
    b, c, h, w = x.shape
    rh = int(round(_H_RATIO * h))
    hw = h * w
    x3 = x.reshape(b, c, hw)

    mesh = plsc.VectorSubcoreMesh(
        core_axis_name="c", subcore_axis_name="s",
        num_cores=_NC, num_subcores=_NS)
    body = functools.partial(_sc_body, b=b, c=c, h=h, w=w, rh=rh)
    run = pl.kernel(
        body,
        out_type=jax.ShapeDtypeStruct((b, c, hw), x.dtype),
        mesh=mesh,
        scratch_types=[
            pltpu.VMEM((_CC, hw), jnp.float32),
            pltpu.VMEM((hw // _L, _L), jnp.float32),
            pltpu.SMEM((h,), jnp.float32),
        ],
    )
    out = run(x3)
    return out.reshape(b, c, h, w)


# speedup vs baseline: 1.5805x; 1.5135x over previous
import functools

import jax
import jax.numpy as jnp
from jax import lax
from jax.experimental import pallas as pl
from jax.experimental.pallas import tpu as pltpu

_H_RATIO = 0.33


def _tree_sum(parts):
    while len(parts) > 1:
        nxt = [a + b for a, b in zip(parts[::2], parts[1::2])]
        if len(parts) % 2:
            nxt.append(parts[-1])
        parts = nxt
    return parts[0]


def _bdt_block(x_ref, o_ref, *, h, w, rh):
    xb = x_ref[...]
    s_blk, c, hw = xb.shape
    pad = 256
    ngrp = pad // w

    nchunk = 8
    step = c // nchunk
    parts = [
        jnp.sum(xb[:, i * step:(i + 1) * step, :] ** 2, axis=1)
        for i in range(nchunk)
    ]
    e = _tree_sum(parts)

    e = jnp.concatenate(
        [e, jnp.full((s_blk, pad - hw), -1.0, e.dtype)], axis=1)

    lane = lax.broadcasted_iota(jnp.int32, (s_blk, pad), 1)

    m = e
    s = 1
    while s < w:
        up = pltpu.roll(m, pad - s, axis=1)
        dn = pltpu.roll(m, s, axis=1)
        m = jnp.maximum(m, jnp.where((lane % (2 * s)) < s, up, dn))
        s *= 2

    g = lane // w
    beats = []
    for d in range(1, ngrp):
        md = pltpu.roll(m, pad - w * d, axis=1)
        gd = g + d
        gd = jnp.where(gd >= ngrp, gd - ngrp, gd)
        beat = (md > m) | ((md == m) & (gd > g))
        beats.append(beat.astype(jnp.int32))
    rank = _tree_sum(beats)

    keep = (rank >= rh).astype(xb.dtype)[:, :hw]
    o_ref[:, :, 0:hw] = xb * keep[:, None, :]


def kernel(x):
    b, c, h, w = x.shape
    rh = int(round(_H_RATIO * h))
    hw = h * w
    s_blk = 4
    x3 = x.reshape(b, c, hw)

    body = functools.partial(_bdt_block, h=h, w=w, rh=rh)
    out = pl.pallas_call(
        body,
        grid=(b // s_blk,),
        in_specs=[pl.BlockSpec((s_blk, c, hw), lambda i: (i, 0, 0))],
        out_specs=pl.BlockSpec((s_blk, c, 256), lambda i: (i, 0, 0)),
        out_shape=jax.ShapeDtypeStruct((b, c, 256), x.dtype),
    )(x3)
    return out[:, :, :hw].reshape(b, c, h, w)
